# baseline (speedup 1.0000x reference)
import functools

import jax
import jax.numpy as jnp
from jax import lax
from jax.experimental import pallas as pl
from jax.experimental.pallas import tpu as pltpu
from jax.experimental.pallas import tpu_sc as plsc

VOCAB = 100000
SEQ = 200
EMB = 128
BATCH = 1024

NC = 2
NS = 16
NW = NC * NS
SEQ_PER_W = BATCH // NW
HALF = SEQ // 2

_mesh = plsc.VectorSubcoreMesh(
    core_axis_name="c", subcore_axis_name="s", num_cores=NC, num_subcores=NS
)


@functools.partial(
    pl.kernel,
    out_type=jax.ShapeDtypeStruct((BATCH * SEQ, EMB), jnp.float32),
    mesh=_mesh,
    scratch_types=[
        pltpu.VMEM((2 * SEQ_PER_W, HALF), jnp.int32),
        pltpu.VMEM((SEQ, EMB), jnp.float32),
        pltpu.VMEM((2, SEQ, EMB), jnp.float32),
        pltpu.SemaphoreType.DMA,
        pltpu.SemaphoreType.DMA,
    ],
)
def _emb_kernel(
    idx_hbm, table_hbm, pos_hbm, out_hbm, idx_v, pos_v, buf, gsem0, gsem1
):
    gsem = (gsem0, gsem1)
    wid = lax.axis_index("s") * NC + lax.axis_index("c")
    pltpu.sync_copy(idx_hbm.at[wid], idx_v)
    pltpu.sync_copy(pos_hbm, pos_v)

    def start_gather(seq, b):
        g0 = pltpu.async_copy(
            table_hbm.at[idx_v.at[2 * seq]],
            buf.at[b, pl.ds(0, HALF)],
            gsem[b],
            add=True,
        )
        g1 = pltpu.async_copy(
            table_hbm.at[idx_v.at[2 * seq + 1]],
            buf.at[b, pl.ds(HALF, HALF)],
            gsem[b],
            add=True,
        )
        return g0, g1

    @pl.loop(0, SEQ_PER_W // 2)
    def _pair_loop(i):
        for b in range(2):
            pltpu.sync_copy(pos_hbm, buf.at[b])
        gathers = [start_gather(2 * i + b, b) for b in range(2)]
        for b in range(2):
            seq = 2 * i + b
            g0, g1 = gathers[b]
            g0.wait()
            g1.wait()
            base = wid * (SEQ_PER_W * SEQ) + seq * SEQ
            pltpu.sync_copy(buf.at[b], out_hbm.at[pl.ds(base, SEQ)])


def kernel(x, token_table, pos_table):
    idx = x.reshape(NW, 2 * SEQ_PER_W, HALF)
    out = _emb_kernel(idx, token_table, pos_table)
    return out.reshape(BATCH, SEQ, EMB)


# speedup vs baseline: 1.4910x; 1.4910x over previous
import functools

import jax
import jax.numpy as jnp
from jax import lax
from jax.experimental import pallas as pl
from jax.experimental.pallas import tpu as pltpu
from jax.experimental.pallas import tpu_sc as plsc

VOCAB = 100000
SEQ = 200
EMB = 128
BATCH = 1024

NC = 2
NS = 16
NW = NC * NS
SEQ_PER_W = BATCH // NW
HALF = SEQ // 2
NBUF = 4

_mesh = plsc.VectorSubcoreMesh(
    core_axis_name="c", subcore_axis_name="s", num_cores=NC, num_subcores=NS
)


@functools.partial(
    pl.kernel,
    out_type=jax.ShapeDtypeStruct((BATCH * SEQ, EMB), jnp.float32),
    mesh=_mesh,
    scratch_types=[
        pltpu.VMEM((2 * NBUF, HALF), jnp.int32),
        pltpu.VMEM((SEQ, EMB), jnp.float32),
        pltpu.VMEM((NBUF, SEQ, EMB), jnp.float32),
        pltpu.SemaphoreType.DMA,
        pltpu.SemaphoreType.DMA,
        pltpu.SemaphoreType.DMA,
        pltpu.SemaphoreType.DMA,
    ],
)
def _emb_kernel(idx_hbm, table_hbm, pos_hbm, out_hbm, idx_v, pos_v, buf, *gsem):
    wid = lax.axis_index("s") * NC + lax.axis_index("c")
    pltpu.sync_copy(pos_hbm, pos_v)

    def start_gather(k):
        g0 = pltpu.async_copy(
            table_hbm.at[idx_v.at[2 * k]], buf.at[k, pl.ds(0, HALF)], gsem[k]
        )
        g1 = pltpu.async_copy(
            table_hbm.at[idx_v.at[2 * k + 1]], buf.at[k, pl.ds(HALF, HALF)], gsem[k]
        )
        return g0, g1

    @pl.loop(0, SEQ_PER_W // NBUF)
    def _ring_loop(i):
        pltpu.sync_copy(idx_hbm.at[wid, pl.ds(2 * NBUF * i, 2 * NBUF)], idx_v)
        gathers = [start_gather(k) for k in range(NBUF)]
        for k in range(NBUF):
            seq = NBUF * i + k
            g0, g1 = gathers[k]
            g0.wait()
            g1.wait()

            @pl.loop(0, SEQ, unroll=4)
            def _row_loop(r):
                for j in range(EMB // 16):
                    sl = pl.ds(j * 16, 16)
                    plsc.addupdate(buf.at[k, r, sl], pos_v[r, sl])

            base = wid * (SEQ_PER_W * SEQ) + seq * SEQ
            pltpu.sync_copy(buf.at[k], out_hbm.at[pl.ds(base, SEQ)])


def kernel(x, token_table, pos_table):
    idx = x.reshape(NW, 2 * SEQ_PER_W, HALF)
    out = _emb_kernel(idx, token_table, pos_table)
    return out.reshape(BATCH, SEQ, EMB)


# speedup vs baseline: 1.8353x; 1.2309x over previous
import functools

import jax
import jax.numpy as jnp
from jax import lax
from jax.experimental import pallas as pl
from jax.experimental.pallas import tpu as pltpu
from jax.experimental.pallas import tpu_sc as plsc

VOCAB = 100000
SEQ = 200
EMB = 128
BATCH = 1024

NC = 2
NS = 16
NW = NC * NS
SEQ_PER_W = BATCH // NW
HALF = SEQ // 2
NBUF = 4

_mesh = plsc.VectorSubcoreMesh(
    core_axis_name="c", subcore_axis_name="s", num_cores=NC, num_subcores=NS
)


@functools.partial(
    pl.kernel,
    out_type=jax.ShapeDtypeStruct((BATCH * SEQ, EMB), jnp.float32),
    mesh=_mesh,
    scratch_types=[
        pltpu.VMEM((2 * NBUF, HALF), jnp.int32),
        pltpu.VMEM((SEQ, EMB), jnp.float32),
        pltpu.VMEM((NBUF, SEQ, EMB), jnp.float32),
        pltpu.SemaphoreType.DMA,
        pltpu.SemaphoreType.DMA,
        pltpu.SemaphoreType.DMA,
        pltpu.SemaphoreType.DMA,
        pltpu.SemaphoreType.DMA,
    ],
)
def _emb_kernel(idx_hbm, table_hbm, pos_hbm, out_hbm, idx_v, pos_v, buf, *sems):
    gsem = sems[:4]
    wsem = sems[4]
    wid = lax.axis_index("s") * NC + lax.axis_index("c")
    pltpu.sync_copy(pos_hbm, pos_v)

    def start_gather(k):
        g0 = pltpu.async_copy(
            table_hbm.at[idx_v.at[2 * k]], buf.at[k, pl.ds(0, HALF)], gsem[k]
        )
        g1 = pltpu.async_copy(
            table_hbm.at[idx_v.at[2 * k + 1]], buf.at[k, pl.ds(HALF, HALF)], gsem[k]
        )
        return g0, g1

    @pl.loop(0, SEQ_PER_W // NBUF)
    def _ring_loop(i):
        pltpu.sync_copy(idx_hbm.at[wid, pl.ds(2 * NBUF * i, 2 * NBUF)], idx_v)
        gathers = [start_gather(k) for k in range(NBUF)]
        writebacks = []
        for p in range(NBUF // 2):
            k0, k1 = 2 * p, 2 * p + 1
            for g in gathers[k0] + gathers[k1]:
                g.wait()

            @pl.loop(0, SEQ, unroll=2)
            def _row_loop(r):
                for j in range(EMB // 16):
                    sl = pl.ds(j * 16, 16)
                    v = pos_v[r, sl]
                    plsc.addupdate(buf.at[k0, r, sl], v)
                    plsc.addupdate(buf.at[k1, r, sl], v)

            for k in (k0, k1):
                base = wid * (SEQ_PER_W * SEQ) + (NBUF * i + k) * SEQ
                writebacks.append(
                    pltpu.async_copy(buf.at[k], out_hbm.at[pl.ds(base, SEQ)], wsem)
                )
        for w in writebacks:
            w.wait()


def kernel(x, token_table, pos_table):
    idx = x.reshape(NW, 2 * SEQ_PER_W, HALF)
    out = _emb_kernel(idx, token_table, pos_table)
    return out.reshape(BATCH, SEQ, EMB)


# speedup vs baseline: 2.0082x; 1.0942x over previous
import functools

import jax
import jax.numpy as jnp
from jax import lax
from jax.experimental import pallas as pl
from jax.experimental.pallas import tpu as pltpu
from jax.experimental.pallas import tpu_sc as plsc

VOCAB = 100000
SEQ = 200
EMB = 128
BATCH = 1024

NC = 2
NS = 16
NW = NC * NS
SEQ_PER_W = BATCH // NW
HALF = SEQ // 2
NBUF = 4
WAVES = 2

_mesh = plsc.VectorSubcoreMesh(
    core_axis_name="c", subcore_axis_name="s", num_cores=NC, num_subcores=NS
)


@functools.partial(
    pl.kernel,
    out_type=jax.ShapeDtypeStruct((BATCH * SEQ, EMB), jnp.float32),
    mesh=_mesh,
    scratch_types=[
        pltpu.VMEM((2 * NBUF * WAVES, HALF), jnp.int32),
        pltpu.VMEM((SEQ, EMB), jnp.float32),
        pltpu.VMEM((NBUF, SEQ, EMB), jnp.float32),
        pltpu.SemaphoreType.DMA,
        pltpu.SemaphoreType.DMA,
        pltpu.SemaphoreType.DMA,
        pltpu.SemaphoreType.DMA,
        pltpu.SemaphoreType.DMA,
        pltpu.SemaphoreType.DMA,
        pltpu.SemaphoreType.DMA,
        pltpu.SemaphoreType.DMA,
    ],
)
def _emb_kernel(idx_hbm, table_hbm, pos_hbm, out_hbm, idx_v, pos_v, buf, *sems):
    gsem = sems[:NBUF]
    wsem = sems[NBUF:]
    wid = lax.axis_index("s") * NC + lax.axis_index("c")
    pltpu.sync_copy(pos_hbm, pos_v)

    def start_gather(row, k):
        g0 = pltpu.async_copy(
            table_hbm.at[idx_v.at[row]], buf.at[k, pl.ds(0, HALF)], gsem[k]
        )
        g1 = pltpu.async_copy(
            table_hbm.at[idx_v.at[row + 1]], buf.at[k, pl.ds(HALF, HALF)], gsem[k]
        )
        return g0, g1

    def process_wave(i, wave, gathers):
        wbs = []
        for p in range(NBUF // 2):
            k0, k1 = 2 * p, 2 * p + 1
            for g in gathers[k0] + gathers[k1]:
                g.wait()

            @pl.loop(0, SEQ, unroll=2)
            def _row_loop(r):
                for j in range(EMB // 16):
                    sl = pl.ds(j * 16, 16)
                    v = pos_v[r, sl]
                    plsc.addupdate(buf.at[k0, r, sl], v)
                    plsc.addupdate(buf.at[k1, r, sl], v)

            for k in (k0, k1):
                seq = NBUF * WAVES * i + NBUF * wave + k
                base = wid * (SEQ_PER_W * SEQ) + seq * SEQ
                wbs.append(
                    pltpu.async_copy(
                        buf.at[k], out_hbm.at[pl.ds(base, SEQ)], wsem[k]
                    )
                )
        return wbs

    @pl.loop(0, SEQ_PER_W // (NBUF * WAVES))
    def _body(i):
        pltpu.sync_copy(
            idx_hbm.at[wid, pl.ds(2 * NBUF * WAVES * i, 2 * NBUF * WAVES)], idx_v
        )
        gathers = [start_gather(2 * k, k) for k in range(NBUF)]
        wbs = process_wave(i, 0, gathers)
        gathers2 = []
        for k in range(NBUF):
            wbs[k].wait()
            gathers2.append(start_gather(2 * NBUF + 2 * k, k))
        wbs2 = process_wave(i, 1, gathers2)
        for w in wbs2:
            w.wait()


def kernel(x, token_table, pos_table):
    idx = x.reshape(NW, 2 * SEQ_PER_W, HALF)
    out = _emb_kernel(idx, token_table, pos_table)
    return out.reshape(BATCH, SEQ, EMB)


# speedup vs baseline: 2.1163x; 1.0538x over previous
import functools

import jax
import jax.numpy as jnp
from jax import lax
from jax.experimental import pallas as pl
from jax.experimental.pallas import tpu as pltpu
from jax.experimental.pallas import tpu_sc as plsc

VOCAB = 100000
SEQ = 200
EMB = 128
BATCH = 1024

NC = 2
NS = 16
NW = NC * NS
SEQ_PER_W = BATCH // NW
HALF = SEQ // 2
NBUF = 4
WAVES = 4

_mesh = plsc.VectorSubcoreMesh(
    core_axis_name="c", subcore_axis_name="s", num_cores=NC, num_subcores=NS
)


@functools.partial(
    pl.kernel,
    out_type=jax.ShapeDtypeStruct((BATCH * SEQ, EMB), jnp.float32),
    mesh=_mesh,
    scratch_types=[
        pltpu.VMEM((2, 2 * NBUF, HALF), jnp.int32),
        pltpu.VMEM((SEQ, EMB), jnp.float32),
        pltpu.VMEM((NBUF, SEQ, EMB), jnp.float32),
        pltpu.SemaphoreType.DMA,
        pltpu.SemaphoreType.DMA,
        pltpu.SemaphoreType.DMA,
        pltpu.SemaphoreType.DMA,
        pltpu.SemaphoreType.DMA,
        pltpu.SemaphoreType.DMA,
        pltpu.SemaphoreType.DMA,
        pltpu.SemaphoreType.DMA,
    ],
)
def _emb_kernel(idx_hbm, table_hbm, pos_hbm, out_hbm, idx_v, pos_v, buf, *sems):
    gsem = sems[:NBUF]
    wsem = sems[NBUF:]
    wid = lax.axis_index("s") * NC + lax.axis_index("c")
    pltpu.sync_copy(pos_hbm, pos_v)

    def start_gather(slot, k):
        g0 = pltpu.async_copy(
            table_hbm.at[idx_v.at[slot, 2 * k]], buf.at[k, pl.ds(0, HALF)], gsem[k]
        )
        g1 = pltpu.async_copy(
            table_hbm.at[idx_v.at[slot, 2 * k + 1]],
            buf.at[k, pl.ds(HALF, HALF)],
            gsem[k],
        )
        return g0, g1

    def stage_idx(i, wave):
        pltpu.sync_copy(
            idx_hbm.at[wid, pl.ds(2 * NBUF * (WAVES * i + wave), 2 * NBUF)],
            idx_v.at[wave % 2],
        )

    def process_wave(i, wave, gathers):
        wbs = []
        for p in range(NBUF // 2):
            k0, k1 = 2 * p, 2 * p + 1
            for g in gathers[k0] + gathers[k1]:
                g.wait()

            @pl.loop(0, SEQ, unroll=2)
            def _row_loop(r):
                for j in range(EMB // 16):
                    sl = pl.ds(j * 16, 16)
                    v = pos_v[r, sl]
                    plsc.addupdate(buf.at[k0, r, sl], v)
                    plsc.addupdate(buf.at[k1, r, sl], v)

            for k in (k0, k1):
                seq = NBUF * WAVES * i + NBUF * wave + k
                base = wid * (SEQ_PER_W * SEQ) + seq * SEQ
                wbs.append(
                    pltpu.async_copy(
                        buf.at[k], out_hbm.at[pl.ds(base, SEQ)], wsem[k]
                    )
                )
        return wbs

    @pl.loop(0, SEQ_PER_W // (NBUF * WAVES))
    def _body(i):
        stage_idx(i, 0)
        gathers = [start_gather(0, k) for k in range(NBUF)]
        for wave in range(WAVES):
            if wave + 1 < WAVES:
                stage_idx(i, wave + 1)
            wbs = process_wave(i, wave, gathers)
            if wave + 1 < WAVES:
                gathers = []
                for k in range(NBUF):
                    wbs[k].wait()
                    gathers.append(start_gather((wave + 1) % 2, k))
            else:
                for w in wbs:
                    w.wait()


def kernel(x, token_table, pos_table):
    idx = x.reshape(NW, 2 * SEQ_PER_W, HALF)
    out = _emb_kernel(idx, token_table, pos_table)
    return out.reshape(BATCH, SEQ, EMB)


# speedup vs baseline: 2.1677x; 1.0243x over previous
import functools

import jax
import jax.numpy as jnp
from jax import lax
from jax.experimental import pallas as pl
from jax.experimental.pallas import tpu as pltpu
from jax.experimental.pallas import tpu_sc as plsc

VOCAB = 100000
SEQ = 200
EMB = 128
BATCH = 1024

NC = 2
NS = 16
NW = NC * NS
SEQ_PER_W = BATCH // NW
HALF = SEQ // 2
NBUF = 4
WAVES = 8

_mesh = plsc.VectorSubcoreMesh(
    core_axis_name="c", subcore_axis_name="s", num_cores=NC, num_subcores=NS
)


@functools.partial(
    pl.kernel,
    out_type=jax.ShapeDtypeStruct((BATCH * SEQ, EMB), jnp.float32),
    mesh=_mesh,
    scratch_types=[
        pltpu.VMEM((2, 2 * NBUF, HALF), jnp.int32),
        pltpu.VMEM((SEQ, EMB), jnp.float32),
        pltpu.VMEM((NBUF, SEQ, EMB), jnp.float32),
        pltpu.SemaphoreType.DMA,
        pltpu.SemaphoreType.DMA,
        pltpu.SemaphoreType.DMA,
        pltpu.SemaphoreType.DMA,
        pltpu.SemaphoreType.DMA,
        pltpu.SemaphoreType.DMA,
        pltpu.SemaphoreType.DMA,
        pltpu.SemaphoreType.DMA,
    ],
)
def _emb_kernel(idx_hbm, table_hbm, pos_hbm, out_hbm, idx_v, pos_v, buf, *sems):
    gsem = sems[:NBUF]
    wsem = sems[NBUF:]
    wid = lax.axis_index("s") * NC + lax.axis_index("c")
    pltpu.sync_copy(pos_hbm, pos_v)

    def start_gather(slot, k):
        g0 = pltpu.async_copy(
            table_hbm.at[idx_v.at[slot, 2 * k]], buf.at[k, pl.ds(0, HALF)], gsem[k]
        )
        g1 = pltpu.async_copy(
            table_hbm.at[idx_v.at[slot, 2 * k + 1]],
            buf.at[k, pl.ds(HALF, HALF)],
            gsem[k],
        )
        return g0, g1

    def stage_idx(i, wave):
        pltpu.sync_copy(
            idx_hbm.at[wid, pl.ds(2 * NBUF * (WAVES * i + wave), 2 * NBUF)],
            idx_v.at[wave % 2],
        )

    def process_wave(i, wave, gathers):
        wbs = []
        for p in range(NBUF // 2):
            k0, k1 = 2 * p, 2 * p + 1
            for g in gathers[k0] + gathers[k1]:
                g.wait()

            @pl.loop(0, SEQ, unroll=2)
            def _row_loop(r):
                for j in range(EMB // 16):
                    sl = pl.ds(j * 16, 16)
                    v = pos_v[r, sl]
                    plsc.addupdate(buf.at[k0, r, sl], v)
                    plsc.addupdate(buf.at[k1, r, sl], v)

            for k in (k0, k1):
                seq = NBUF * WAVES * i + NBUF * wave + k
                base = wid * (SEQ_PER_W * SEQ) + seq * SEQ
                wbs.append(
                    pltpu.async_copy(
                        buf.at[k], out_hbm.at[pl.ds(base, SEQ)], wsem[k]
                    )
                )
        return wbs

    @pl.loop(0, SEQ_PER_W // (NBUF * WAVES))
    def _body(i):
        stage_idx(i, 0)
        gathers = [start_gather(0, k) for k in range(NBUF)]
        for wave in range(WAVES):
            if wave + 1 < WAVES:
                stage_idx(i, wave + 1)
            wbs = process_wave(i, wave, gathers)
            if wave + 1 < WAVES:
                gathers = []
                for k in range(NBUF):
                    wbs[k].wait()
                    gathers.append(start_gather((wave + 1) % 2, k))
            else:
                for w in wbs:
                    w.wait()


def kernel(x, token_table, pos_table):
    idx = x.reshape(NW, 2 * SEQ_PER_W, HALF)
    out = _emb_kernel(idx, token_table, pos_table)
    return out.reshape(BATCH, SEQ, EMB)
